# baseline (speedup 1.0000x reference)
import functools

import jax
import jax.numpy as jnp
from jax import lax
from jax.experimental import pallas as pl
from jax.experimental.pallas import tpu as pltpu
from jax.experimental.pallas import tpu_sc as plsc

from jax._src.pallas.mosaic import lowering as _tc_lowering


def _reshape_memref_reinterpret(ref, reshaper, ref_aval, ref_block_shape):
    ref_ty = _tc_lowering.ir.MemRefType(ref.type)
    dims = "x".join(str(s) for s in reshaper.shape)
    elt = str(ref_ty.element_type)
    target_ty = _tc_lowering.ir.Type.parse(
        f"memref<{dims}x{elt}, #tpu.tiled<(128),[1]>, {ref_ty.memory_space}>",
        ref_ty.context,
    )
    return _tc_lowering.tpu.reinterpret_cast(target_ty, ref), reshaper.shape


_tc_lowering._reshape_memref = _reshape_memref_reinterpret


_orig_slice_memref = _tc_lowering._slice_memref


def _slice_memref_keep_layout(ref, indexer, ref_aval, ref_block_shape):
    ref_ty = _tc_lowering.ir.MemRefType(ref.type)
    layout = str(ref_ty.layout)
    if ("hbm" in str(ref_ty.memory_space)
            and "tiled" in layout
            and len(ref_ty.shape) == 1
            and indexer.get_indexer_shape() == tuple(ref_ty.shape)):
        return ref, ref_block_shape
    return _orig_slice_memref(ref, indexer, ref_aval, ref_block_shape)


_tc_lowering._slice_memref = _slice_memref_keep_layout

B = 16384
NF = 26
ND = 13
VOCAB = 1000000
NW = 32
RPW = B // NW
NV = RPW // 16
EPW = NF * RPW
NG = 26
GSZ = EPW // NG

CT = (VOCAB + 127) // 128

_mesh = plsc.VectorSubcoreMesh(core_axis_name="c", subcore_axis_name="s")


@functools.partial(
    pl.kernel,
    mesh=_mesh,
    out_type=jax.ShapeDtypeStruct((B,), jnp.float32),
    compiler_params=pltpu.CompilerParams(needs_layout_passes=False),
    scratch_types=[
        pltpu.VMEM((EPW,), jnp.int32),
        pltpu.VMEM((EPW,), jnp.float32),
        pltpu.VMEM((ND * RPW,), jnp.float32),
        pltpu.VMEM((ND * 16,), jnp.float32),
        pltpu.VMEM((RPW,), jnp.float32),
        pltpu.SemaphoreType.DMA,
    ],
)
def _linear_sc(idx_hbm, xd_hbm, table_hbm, w_hbm, out_hbm,
               idx_v, gat_v, xd_v, w_v, out_v, sem):
    wid = lax.axis_index("s") * 2 + lax.axis_index("c")
    base = wid * RPW

    pltpu.sync_copy(idx_hbm.at[pl.ds(wid * EPW, EPW)], idx_v)

    table_flat = table_hbm.reshape(NF * VOCAB)

    copies = []
    for g in range(NG):
        sl = pl.ds(g * GSZ, GSZ)
        copies.append(
            pltpu.async_copy(table_flat.at[idx_v.at[sl]], gat_v.at[sl], sem))

    pltpu.sync_copy(xd_hbm.at[pl.ds(wid * (ND * RPW), ND * RPW)], xd_v)
    pltpu.sync_copy(w_hbm, w_v)

    for c in copies:
        c.wait()

    w_bc = [w_v[pl.ds(d * 16, 16)] for d in range(ND)]

    i26 = lax.iota(jnp.int32, 16) * NF
    i13 = lax.iota(jnp.int32, 16) * ND

    def reduce(j, carry):
        eb = j * 16 * NF
        db = j * 16 * ND
        acc = plsc.load_gather(gat_v, [i26 + eb])
        for f in range(1, NF):
            acc = acc + plsc.load_gather(gat_v, [i26 + (eb + f)])
        for d in range(ND):
            acc = acc + plsc.load_gather(xd_v, [i13 + (db + d)]) * w_bc[d]
        out_v[pl.ds(j * 16, 16)] = acc
        return carry

    lax.fori_loop(0, NV, reduce, 0)

    pltpu.sync_copy(out_v, out_hbm.at[pl.ds(base, RPW)])


def kernel(X, emb_tables, dense_weight):
    ids = X[:, :NF].astype(jnp.int32)
    f = jnp.arange(NF, dtype=jnp.int32)
    fo = ((f >> 3) * CT) * 1024 + (f & 7) * 128
    phys = (fo[None, :]
            + ((ids >> 7) << 10)
            + (ids & 127)).reshape(-1)
    xd = X[:, NF:].reshape(-1)
    w = jnp.broadcast_to(dense_weight, (ND, 16)).reshape(-1)
    out = _linear_sc(phys, xd, emb_tables, w)
    return out[:, None]


# speedup vs baseline: 1.0073x; 1.0073x over previous
import functools

import jax
import jax.numpy as jnp
from jax import lax
from jax.experimental import pallas as pl
from jax.experimental.pallas import tpu as pltpu
from jax.experimental.pallas import tpu_sc as plsc

from jax._src.pallas.mosaic import lowering as _tc_lowering


def _reshape_memref_reinterpret(ref, reshaper, ref_aval, ref_block_shape):
    ref_ty = _tc_lowering.ir.MemRefType(ref.type)
    dims = "x".join(str(s) for s in reshaper.shape)
    elt = str(ref_ty.element_type)
    if len(reshaper.shape) == 1:
        target_ty = _tc_lowering.ir.Type.parse(
            f"memref<{dims}x{elt}, #tpu.tiled<(128),[1]>,"
            f" {ref_ty.memory_space}>",
            ref_ty.context,
        )
    else:
        target_ty = _tc_lowering.ir.Type.parse(
            f"memref<{dims}x{elt}, #tpu.tiled<({reshaper.shape[-1]}),[1,1]>,"
            f" {ref_ty.memory_space}>",
            ref_ty.context,
        )
    return _tc_lowering.tpu.reinterpret_cast(target_ty, ref), reshaper.shape


_tc_lowering._reshape_memref = _reshape_memref_reinterpret


_orig_slice_memref = _tc_lowering._slice_memref


def _slice_memref_keep_layout(ref, indexer, ref_aval, ref_block_shape):
    ref_ty = _tc_lowering.ir.MemRefType(ref.type)
    layout = str(ref_ty.layout)
    if ("hbm" in str(ref_ty.memory_space)
            and "tiled" in layout):
        if indexer.get_indexer_shape() == tuple(ref_ty.shape):
            return ref, ref_block_shape
        if len(ref_ty.shape) != 1:
            return _orig_slice_memref(ref, indexer, ref_aval, ref_block_shape)
        starts, sizes, strides, squeeze_dims, out_block_shape = (
            _tc_lowering._indexer_to_start_size_stride(
                indexer, ref_block_shape, cast_to_index=False))
        assert all((s is None or s == 1) for s in strides)
        assert not any(squeeze_dims)
        static_sizes = []
        dynamic_sizes = []
        for s in sizes:
            if not isinstance(s, _tc_lowering.ir.Value):
                static_sizes.append(s)
            else:
                static_sizes.append(
                    _tc_lowering.ir.ShapedType.get_dynamic_size())
                dynamic_sizes.append(s)
        dims = "x".join(str(s) for s in static_sizes)
        target_ty = _tc_lowering.ir.Type.parse(
            f"memref<{dims}x{ref_ty.element_type}, {layout},"
            f" {ref_ty.memory_space}>",
            ref_ty.context,
        )
        out = _tc_lowering.tpu.memref_slice(
            target_ty, ref, starts, dynamic_sizes)
        return out, out_block_shape
    return _orig_slice_memref(ref, indexer, ref_aval, ref_block_shape)


_tc_lowering._slice_memref = _slice_memref_keep_layout

B = 16384
NF = 26
ND = 13
NX = NF + ND
VOCAB = 1000000
NW = 32
RPW = B // NW
NV = RPW // 16
XBLK = RPW * 128
CT = (VOCAB + 127) // 128

_mesh = plsc.VectorSubcoreMesh(core_axis_name="c", subcore_axis_name="s")


@functools.partial(
    pl.kernel,
    mesh=_mesh,
    out_type=jax.ShapeDtypeStruct((B,), jnp.float32),
    compiler_params=pltpu.CompilerParams(needs_layout_passes=False),
    scratch_types=[
        pltpu.VMEM((XBLK,), jnp.float32),
        pltpu.VMEM((XBLK // 16,), jnp.int32),
        pltpu.VMEM((NF * RPW,), jnp.int32),
        pltpu.VMEM((NF * RPW,), jnp.float32),
        pltpu.VMEM((ND * 16,), jnp.float32),
        pltpu.VMEM((RPW,), jnp.float32),
        pltpu.SemaphoreType.DMA,
    ],
)
def _linear_sc(x_hbm, table_hbm, w_hbm, out_hbm,
               xblk_v, ramp_v, idx_v, gat_v, w_v, out_v, sem):
    wid = lax.axis_index("s") * 2 + lax.axis_index("c")
    base = wid * RPW
    ngr = XBLK // 16

    x16 = x_hbm.reshape(B * NX // 16, 16)
    table_flat = table_hbm.reshape(NF * VOCAB)

    i16 = lax.iota(jnp.int32, 16)

    def fill_ramp(k, carry):
        ramp_v[pl.ds(k * 16, 16)] = i16 + (wid * ngr + k * 16)
        return carry

    lax.fori_loop(0, ngr // 16, fill_ramp, 0)
    pltpu.async_copy(x16.at[ramp_v], xblk_v.reshape(ngr, 16), sem).wait()
    pltpu.sync_copy(w_hbm, w_v)

    pat = ((i16 >> 3) << 10) + ((i16 & 7) << 7)

    def conv(j, carry):
        for f in range(NF):
            foff = ((f >> 3) * CT) * 1024 + (f & 7) * 128
            ids = plsc.load_gather(
                xblk_v, [pat + (j * 2048 + f)]).astype(jnp.int32)
            idx_v[pl.ds(f * RPW + j * 16, 16)] = (
                foff + ((ids >> 7) << 10) + (ids & 127))
        return carry

    lax.fori_loop(0, NV, conv, 0)

    copies = []
    for f in range(NF):
        sl = pl.ds(f * RPW, RPW)
        copies.append(
            pltpu.async_copy(table_flat.at[idx_v.at[sl]], gat_v.at[sl], sem))

    for c in copies:
        c.wait()

    w_bc = [w_v[pl.ds(d * 16, 16)] for d in range(ND)]

    def reduce(j, carry):
        acc = gat_v[pl.ds(j * 16, 16)]
        for f in range(1, NF):
            acc = acc + gat_v[pl.ds(f * RPW + j * 16, 16)]
        for d in range(ND):
            acc = acc + plsc.load_gather(
                xblk_v, [pat + (j * 2048 + NF + d)]) * w_bc[d]
        out_v[pl.ds(j * 16, 16)] = acc
        return carry

    lax.fori_loop(0, NV, reduce, 0)

    pltpu.sync_copy(out_v, out_hbm.at[pl.ds(base, RPW)])


def kernel(X, emb_tables, dense_weight):
    w = jnp.broadcast_to(dense_weight, (ND, 16)).reshape(-1)
    out = _linear_sc(X, emb_tables, w)
    return out[:, None]


# speedup vs baseline: 1.4359x; 1.4256x over previous
import functools

import jax
import jax.numpy as jnp
from jax import lax
from jax.experimental import pallas as pl
from jax.experimental.pallas import tpu as pltpu
from jax.experimental.pallas import tpu_sc as plsc

from jax._src.pallas.mosaic import lowering as _tc_lowering


def _reshape_memref_reinterpret(ref, reshaper, ref_aval, ref_block_shape):
    ref_ty = _tc_lowering.ir.MemRefType(ref.type)
    dims = "x".join(str(s) for s in reshaper.shape)
    elt = str(ref_ty.element_type)
    target_ty = _tc_lowering.ir.Type.parse(
        f"memref<{dims}x{elt}, #tpu.tiled<(128),[1]>, {ref_ty.memory_space}>",
        ref_ty.context,
    )
    return _tc_lowering.tpu.reinterpret_cast(target_ty, ref), reshaper.shape


_tc_lowering._reshape_memref = _reshape_memref_reinterpret


_orig_slice_memref = _tc_lowering._slice_memref


def _slice_memref_keep_layout(ref, indexer, ref_aval, ref_block_shape):
    ref_ty = _tc_lowering.ir.MemRefType(ref.type)
    layout = str(ref_ty.layout)
    if ("hbm" in str(ref_ty.memory_space)
            and "tiled" in layout
            and len(ref_ty.shape) == 1
            and indexer.get_indexer_shape() == tuple(ref_ty.shape)):
        return ref, ref_block_shape
    return _orig_slice_memref(ref, indexer, ref_aval, ref_block_shape)


_tc_lowering._slice_memref = _slice_memref_keep_layout

B = 16384
NF = 26
ND = 13
VOCAB = 1000000
NW = 32
RPW = B // NW
NV = RPW // 16

CT = (VOCAB + 127) // 128

_mesh = plsc.VectorSubcoreMesh(core_axis_name="c", subcore_axis_name="s")


@functools.partial(
    pl.kernel,
    mesh=_mesh,
    out_type=jax.ShapeDtypeStruct((B,), jnp.float32),
    scratch_types=[
        pltpu.VMEM((NF * RPW,), jnp.int32),
        pltpu.VMEM((NF * RPW,), jnp.float32),
        pltpu.VMEM((ND * RPW,), jnp.float32),
        pltpu.VMEM((ND * 16,), jnp.float32),
        pltpu.VMEM((RPW,), jnp.float32),
        pltpu.SemaphoreType.DMA,
    ],
)
def _linear_sc(idx_hbm, xd_hbm, table_hbm, w_hbm, out_hbm,
               idx_v, gat_v, xd_v, w_v, out_v, sem):
    wid = lax.axis_index("s") * 2 + lax.axis_index("c")
    base = wid * RPW

    pltpu.sync_copy(idx_hbm.at[pl.ds(wid * (NF * RPW), NF * RPW)], idx_v)

    def to_phys(f, carry):
        k = ((f >> 3) * CT) * 1024 + (f & 7) * 128
        for v in range(NV):
            sl = pl.ds(f * RPW + v * 16, 16)
            ids = idx_v[sl]
            idx_v[sl] = (
                k
                + lax.shift_left(lax.shift_right_logical(ids, 7), 10)
                + lax.bitwise_and(ids, 127)
            )
        return carry

    lax.fori_loop(0, NF, to_phys, 0)

    table_flat = table_hbm.reshape(NF * VOCAB)

    NG = 4
    GSZ = NF * RPW // NG
    copies = []
    for g in range(NG):
        sl = pl.ds(g * GSZ, GSZ)
        copies.append(
            pltpu.async_copy(table_flat.at[idx_v.at[sl]], gat_v.at[sl], sem))

    pltpu.sync_copy(xd_hbm.at[pl.ds(wid * (ND * RPW), ND * RPW)], xd_v)
    pltpu.sync_copy(w_hbm, w_v)

    for c in copies:
        c.wait()

    w_bc = [w_v[pl.ds(d * 16, 16)] for d in range(ND)]

    def reduce(j, carry):
        acc = gat_v[pl.ds(j * 16, 16)]
        for f in range(1, NF):
            acc = acc + gat_v[pl.ds(f * RPW + j * 16, 16)]
        for d in range(ND):
            acc = acc + xd_v[pl.ds(d * RPW + j * 16, 16)] * w_bc[d]
        out_v[pl.ds(j * 16, 16)] = acc
        return carry

    lax.fori_loop(0, NV, reduce, 0)

    pltpu.sync_copy(out_v, out_hbm.at[pl.ds(base, RPW)])


def kernel(X, emb_tables, dense_weight):
    idx = (X[:, :NF].astype(jnp.int32)
           .reshape(NW, RPW, NF).transpose(0, 2, 1).reshape(-1))
    xd = X[:, NF:].reshape(NW, RPW, ND).transpose(0, 2, 1).reshape(-1)
    w = jnp.broadcast_to(dense_weight, (ND, 16)).reshape(-1)
    out = _linear_sc(idx, xd, emb_tables, w)
    return out[:, None]
